# baseline (speedup 1.0000x reference)
import functools

import jax
import jax.numpy as jnp
from jax import lax
from jax.experimental import pallas as pl
from jax.experimental.pallas import tpu as pltpu
from jax.experimental.pallas import tpu_sc as plsc

_NC = 2
_NS = 16
_NW = _NC * _NS

_TAIL_COPIES = ((284, 28), (272, 16), (256, 0))


@functools.lru_cache(maxsize=None)
def _build_sc_kernel(B, V, D, NEG):
    S = 2 + NEG
    S8 = 8
    P = 16
    R8 = S8 * P
    assert B % (_NW * P) == 0
    C = B // (_NW * P)
    PW = B // _NW

    mesh = plsc.VectorSubcoreMesh(core_axis_name="c", subcore_axis_name="s")

    @functools.partial(
        pl.kernel,
        out_type=jax.ShapeDtypeStruct((B, S, D), jnp.float32),
        mesh=mesh,
        scratch_types=[
            pltpu.VMEM((R8,), jnp.int32),
            pltpu.VMEM((R8,), jnp.int32),
            pltpu.VMEM((P,), jnp.int32),
            pltpu.VMEM((R8, D), jnp.float32),
            pltpu.VMEM((R8, 128), jnp.float32),
            pltpu.VMEM((P, 128), jnp.float32),
            pltpu.VMEM((P, 128), jnp.float32),
            pltpu.SemaphoreType.DMA,
        ],
    )
    def sc_kernel(main_idx, tail_idx, t_idx, w_t, w_c, tails, out,
                  mi_v, li_v, ti_v, stage, lbuf, tb0, tb1, sem):
        wid = lax.axis_index("s") * _NC + lax.axis_index("c")

        def chunk(c, carry):
            pltpu.sync_copy(main_idx.at[wid, pl.ds(c * R8, R8)], mi_v)
            pltpu.sync_copy(tail_idx.at[wid, pl.ds(c * R8, R8)], li_v)
            pltpu.sync_copy(t_idx.at[wid, pl.ds(c * P, P)], ti_v)
            g0 = pltpu.async_copy(w_c.at[mi_v, pl.ds(0, 128)],
                                  stage.at[:, pl.ds(0, 128)], sem)
            g1 = pltpu.async_copy(w_c.at[mi_v, pl.ds(128, 128)],
                                  stage.at[:, pl.ds(128, 128)], sem)
            g2 = pltpu.async_copy(tails.at[li_v], lbuf, sem)
            g3 = pltpu.async_copy(w_t.at[ti_v, pl.ds(0, 128)], tb0, sem)
            g4 = pltpu.async_copy(w_t.at[ti_v, pl.ds(128, 128)], tb1, sem)
            g0.wait(); g1.wait(); g2.wait(); g3.wait(); g4.wait()
            for p in range(P):
                for k in range(8):
                    stage[S8 * p, pl.ds(16 * k, 16)] = tb0[p, pl.ds(16 * k, 16)]
                    stage[S8 * p, pl.ds(128 + 16 * k, 16)] = tb1[p, pl.ds(16 * k, 16)]
            for p in range(P):
                for q in range(S):
                    r = S8 * p + q
                    for (d_off, s_off) in _TAIL_COPIES:
                        stage[r, pl.ds(d_off, 16)] = lbuf[r, pl.ds(s_off, 16)]
            b0 = wid * PW + c * P
            for p in range(P):
                pltpu.sync_copy(stage.at[pl.ds(S8 * p, S)], out.at[b0 + p])
            return carry

        lax.fori_loop(0, C, chunk, 0)

    return sc_kernel, S8, C, R8, PW


def kernel(target_words, context_words, negative_examples, W_target, W_context):
    B = target_words.shape[0]
    NEG = negative_examples.shape[1]
    V, D = W_target.shape
    sc_kernel, S8, C, R8, PW = _build_sc_kernel(B, V, D, NEG)

    tw = target_words.astype(jnp.int32)
    cw = context_words.astype(jnp.int32)
    ne = negative_examples.astype(jnp.int32)
    W_t = W_target.astype(jnp.float32)
    W_c = W_context.astype(jnp.float32)

    all8 = jnp.concatenate([tw[:, None], cw[:, None], ne, tw[:, None]], axis=1)
    main_idx = all8.reshape(_NW, C * R8)
    tail_off = jnp.array([V] + [0] * (S8 - 1), jnp.int32)
    tail_idx = (all8 + tail_off[None, :]).reshape(_NW, C * R8)
    t_idx = tw.reshape(_NW, PW)
    tails = jnp.maximum(
        jnp.pad(jnp.concatenate([W_c[:, 256:], W_t[:, 256:]], axis=0),
                ((0, 0), (0, 128 - 44))), -3e38)

    return sc_kernel(main_idx, tail_idx, t_idx, W_t, W_c, tails)


# speedup vs baseline: 1.0097x; 1.0097x over previous
import functools

import jax
import jax.numpy as jnp
from jax import lax
from jax.experimental import pallas as pl
from jax.experimental.pallas import tpu as pltpu
from jax.experimental.pallas import tpu_sc as plsc

_NC = 2
_NS = 16
_NW = _NC * _NS

_TAIL_COPIES = ((284, 28), (272, 16), (256, 0))


@functools.lru_cache(maxsize=None)
def _build_tails_tc(V, D):
    BLK = 4000
    TD = 44
    assert V % BLK == 0

    def body(w_any, o_ref, buf, sem):
        g = pl.program_id(0)
        cp = pltpu.make_async_copy(
            w_any.at[pl.ds(g * BLK, BLK), pl.ds(256, TD)], buf, sem)
        cp.start()
        cp.wait()
        o_ref[:, :TD] = buf[:]
        o_ref[:, TD:] = jnp.zeros((BLK, 128 - TD), jnp.float32)

    return pl.pallas_call(
        body,
        grid=(V // BLK,),
        in_specs=[pl.BlockSpec(memory_space=pltpu.MemorySpace.HBM)],
        out_specs=pl.BlockSpec((BLK, 128), lambda g: (g, 0)),
        out_shape=jax.ShapeDtypeStruct((V, 128), jnp.float32),
        scratch_shapes=[pltpu.VMEM((BLK, TD), jnp.float32),
                        pltpu.SemaphoreType.DMA],
    )


@functools.lru_cache(maxsize=None)
def _build_sc_kernel(B, V, D, NEG):
    S = 2 + NEG
    S8 = 8
    P = 16
    R8 = S8 * P
    assert B % (_NW * P) == 0
    C = B // (_NW * P)
    PW = B // _NW

    mesh = plsc.VectorSubcoreMesh(core_axis_name="c", subcore_axis_name="s")

    @functools.partial(
        pl.kernel,
        out_type=jax.ShapeDtypeStruct((B, S, D), jnp.float32),
        mesh=mesh,
        scratch_types=[
            pltpu.VMEM((R8,), jnp.int32),
            pltpu.VMEM((P,), jnp.int32),
            pltpu.VMEM((R8, D), jnp.float32),
            pltpu.VMEM((R8, 128), jnp.float32),
            pltpu.VMEM((P, 128), jnp.float32),
            pltpu.VMEM((P, 128), jnp.float32),
            pltpu.VMEM((P, 128), jnp.float32),
            pltpu.SemaphoreType.DMA,
        ],
    )
    def sc_kernel(main_idx, t_idx, w_t, w_c, tails_c, tails_t, out,
                  mi_v, ti_v, stage, lbuf, ltbuf, tb0, tb1, sem):
        wid = lax.axis_index("s") * _NC + lax.axis_index("c")

        def chunk(c, carry):
            pltpu.sync_copy(main_idx.at[wid, pl.ds(c * R8, R8)], mi_v)
            pltpu.sync_copy(t_idx.at[wid, pl.ds(c * P, P)], ti_v)
            g0 = pltpu.async_copy(w_c.at[mi_v, pl.ds(0, 128)],
                                  stage.at[:, pl.ds(0, 128)], sem)
            g1 = pltpu.async_copy(w_c.at[mi_v, pl.ds(128, 128)],
                                  stage.at[:, pl.ds(128, 128)], sem)
            g2 = pltpu.async_copy(tails_c.at[mi_v], lbuf, sem)
            g3 = pltpu.async_copy(w_t.at[ti_v, pl.ds(0, 128)], tb0, sem)
            g4 = pltpu.async_copy(w_t.at[ti_v, pl.ds(128, 128)], tb1, sem)
            g5 = pltpu.async_copy(tails_t.at[ti_v], ltbuf, sem)
            g0.wait(); g1.wait(); g2.wait(); g3.wait(); g4.wait(); g5.wait()
            for p in range(P):
                for k in range(8):
                    stage[S8 * p, pl.ds(16 * k, 16)] = tb0[p, pl.ds(16 * k, 16)]
                    stage[S8 * p, pl.ds(128 + 16 * k, 16)] = tb1[p, pl.ds(16 * k, 16)]
            for p in range(P):
                for (d_off, s_off) in _TAIL_COPIES:
                    stage[S8 * p, pl.ds(d_off, 16)] = ltbuf[p, pl.ds(s_off, 16)]
                for q in range(1, S):
                    r = S8 * p + q
                    for (d_off, s_off) in _TAIL_COPIES:
                        stage[r, pl.ds(d_off, 16)] = lbuf[r, pl.ds(s_off, 16)]
            b0 = wid * PW + c * P
            for p in range(P):
                pltpu.sync_copy(stage.at[pl.ds(S8 * p, S)], out.at[b0 + p])
            return carry

        lax.fori_loop(0, C, chunk, 0)

    return sc_kernel, S8, C, R8, PW


def kernel(target_words, context_words, negative_examples, W_target, W_context):
    B = target_words.shape[0]
    NEG = negative_examples.shape[1]
    V, D = W_target.shape
    sc_kernel, S8, C, R8, PW = _build_sc_kernel(B, V, D, NEG)
    tails_tc = _build_tails_tc(V, D)

    tw = target_words.astype(jnp.int32)
    cw = context_words.astype(jnp.int32)
    ne = negative_examples.astype(jnp.int32)
    W_t = W_target.astype(jnp.float32)
    W_c = W_context.astype(jnp.float32)

    all8 = jnp.concatenate([tw[:, None], cw[:, None], ne, tw[:, None]], axis=1)
    main_idx = all8.reshape(_NW, C * R8)
    t_idx = tw.reshape(_NW, PW)
    tails_c = tails_tc(W_c)
    tails_t = tails_tc(W_t)

    return sc_kernel(main_idx, t_idx, W_t, W_c, tails_c, tails_t)


# speedup vs baseline: 1.0252x; 1.0154x over previous
import functools

import jax
import jax.numpy as jnp
from jax import lax
from jax.experimental import pallas as pl
from jax.experimental.pallas import tpu as pltpu
from jax.experimental.pallas import tpu_sc as plsc

_NC = 2
_NS = 16
_NW = _NC * _NS

_TAIL_COPIES = ((284, 28), (272, 16), (256, 0))


@functools.lru_cache(maxsize=None)
def _build_tails_tc(V, D):
    BLK = 4000
    TD = 44
    assert V % BLK == 0

    def body(w_ref, o_ref):
        o_ref[:, :TD] = w_ref[:, D - TD:]
        o_ref[:, TD:] = jnp.zeros((BLK, 128 - TD), jnp.float32)

    return pl.pallas_call(
        body,
        grid=(V // BLK,),
        in_specs=[pl.BlockSpec((BLK, D), lambda g: (g, 0))],
        out_specs=pl.BlockSpec((BLK, 128), lambda g: (g, 0)),
        out_shape=jax.ShapeDtypeStruct((V, 128), jnp.float32),
    )


@functools.lru_cache(maxsize=None)
def _build_sc_kernel(B, V, D, NEG):
    S = 2 + NEG
    S8 = 8
    P = 16
    R8 = S8 * P
    assert B % (_NW * P) == 0
    C = B // (_NW * P)
    PW = B // _NW

    mesh = plsc.VectorSubcoreMesh(core_axis_name="c", subcore_axis_name="s")

    @functools.partial(
        pl.kernel,
        out_type=jax.ShapeDtypeStruct((B, S, D), jnp.float32),
        mesh=mesh,
        scratch_types=[
            pltpu.VMEM((R8,), jnp.int32),
            pltpu.VMEM((P,), jnp.int32),
            pltpu.VMEM((R8, D), jnp.float32),
            pltpu.VMEM((R8, 128), jnp.float32),
            pltpu.VMEM((P, 128), jnp.float32),
            pltpu.VMEM((P, 128), jnp.float32),
            pltpu.VMEM((P, 128), jnp.float32),
            pltpu.SemaphoreType.DMA,
        ],
    )
    def sc_kernel(main_idx, t_idx, w_t, w_c, tails_c, tails_t, out,
                  mi_v, ti_v, stage, lbuf, ltbuf, tb0, tb1, sem):
        wid = lax.axis_index("s") * _NC + lax.axis_index("c")

        def chunk(c, carry):
            pltpu.sync_copy(main_idx.at[wid, pl.ds(c * R8, R8)], mi_v)
            pltpu.sync_copy(t_idx.at[wid, pl.ds(c * P, P)], ti_v)
            g0 = pltpu.async_copy(w_c.at[mi_v, pl.ds(0, 128)],
                                  stage.at[:, pl.ds(0, 128)], sem)
            g1 = pltpu.async_copy(w_c.at[mi_v, pl.ds(128, 128)],
                                  stage.at[:, pl.ds(128, 128)], sem)
            g2 = pltpu.async_copy(tails_c.at[mi_v], lbuf, sem)
            g3 = pltpu.async_copy(w_t.at[ti_v, pl.ds(0, 128)], tb0, sem)
            g4 = pltpu.async_copy(w_t.at[ti_v, pl.ds(128, 128)], tb1, sem)
            g5 = pltpu.async_copy(tails_t.at[ti_v], ltbuf, sem)
            g0.wait(); g1.wait(); g2.wait(); g3.wait(); g4.wait(); g5.wait()
            for p in range(P):
                for k in range(8):
                    stage[S8 * p, pl.ds(16 * k, 16)] = tb0[p, pl.ds(16 * k, 16)]
                    stage[S8 * p, pl.ds(128 + 16 * k, 16)] = tb1[p, pl.ds(16 * k, 16)]
            for p in range(P):
                for (d_off, s_off) in _TAIL_COPIES:
                    stage[S8 * p, pl.ds(d_off, 16)] = ltbuf[p, pl.ds(s_off, 16)]
                for q in range(1, S):
                    r = S8 * p + q
                    for (d_off, s_off) in _TAIL_COPIES:
                        stage[r, pl.ds(d_off, 16)] = lbuf[r, pl.ds(s_off, 16)]
            b0 = wid * PW + c * P
            for p in range(P):
                pltpu.sync_copy(stage.at[pl.ds(S8 * p, S)], out.at[b0 + p])
            return carry

        lax.fori_loop(0, C, chunk, 0)

    return sc_kernel, S8, C, R8, PW


def kernel(target_words, context_words, negative_examples, W_target, W_context):
    B = target_words.shape[0]
    NEG = negative_examples.shape[1]
    V, D = W_target.shape
    sc_kernel, S8, C, R8, PW = _build_sc_kernel(B, V, D, NEG)
    tails_tc = _build_tails_tc(V, D)

    tw = target_words.astype(jnp.int32)
    cw = context_words.astype(jnp.int32)
    ne = negative_examples.astype(jnp.int32)
    W_t = W_target.astype(jnp.float32)
    W_c = W_context.astype(jnp.float32)

    all8 = jnp.concatenate([tw[:, None], cw[:, None], ne, tw[:, None]], axis=1)
    main_idx = all8.reshape(_NW, C * R8)
    t_idx = tw.reshape(_NW, PW)
    tails_c = tails_tc(W_c)
    tails_t = tails_tc(W_t)

    return sc_kernel(main_idx, t_idx, W_t, W_c, tails_c, tails_t)
